# baseline (speedup 1.0000x reference)
import jax
import jax.numpy as jnp
from jax import lax
from jax.experimental import pallas as pl
from jax.experimental.pallas import tpu as pltpu
from jax.experimental.pallas import tpu_sc as plsc

BATCH = 16384
HIST = 200
EMBED_DIM = 64

NUM_CORES = 2
NUM_SUBCORES = 16
NUM_WORKERS = NUM_CORES * NUM_SUBCORES

BPW = BATCH // NUM_WORKERS
NB = 4
STEPS = BPW // NB
NBUF = 2


def _sc_body(actions_hbm, table_hbm, out_hbm, table_sh,
             idx_v, rows_v, sem_g, sem_o, sem_i):
    cid = lax.axis_index("c")
    sid = lax.axis_index("s")
    wid = sid * NUM_CORES + cid
    wbase = wid * BPW

    @pl.when(sid == 0)
    def _():
        pltpu.sync_copy(table_hbm, table_sh)
    plsc.subcore_barrier()

    for b in range(NBUF):
        pltpu.sync_copy(actions_hbm.at[pl.ds(wbase + b * NB, NB)], idx_v[b])

    @pl.loop(0, STEPS // NBUF)
    def _round(t):
        for b in range(NBUF):
            c = t * NBUF + b
            base = wbase + c * NB

            @pl.when(t > 0)
            def _():
                pltpu.make_async_copy(
                    actions_hbm.at[pl.ds(base, NB)], idx_v[b],
                    sem_i[b]).wait()
                pltpu.make_async_copy(
                    rows_v[b], out_hbm.at[pl.ds(base, NB)],
                    sem_o[b]).wait()

            gathers = []
            for j in range(NB):
                gathers.append(pltpu.async_copy(
                    table_sh.at[idx_v[b].at[j]], rows_v[b].at[j],
                    sem_g[b]))
            for g in gathers:
                g.wait()

            pltpu.make_async_copy(
                rows_v[b], out_hbm.at[pl.ds(base, NB)], sem_o[b]).start()

            @pl.when(t < STEPS // NBUF - 1)
            def _():
                pltpu.make_async_copy(
                    actions_hbm.at[pl.ds(base + NBUF * NB, NB)],
                    idx_v[b], sem_i[b]).start()

    for b in range(NBUF):
        c = STEPS - NBUF + b
        pltpu.make_async_copy(
            rows_v[b], out_hbm.at[pl.ds(wbase + c * NB, NB)],
            sem_o[b]).wait()


@jax.jit
def _embed_sc(actions, table8):
    mesh = plsc.VectorSubcoreMesh(core_axis_name="c", subcore_axis_name="s")

    def body(actions_hbm, table_hbm, out_hbm, table_sh, *rest):
        idx_v = rest[0:NBUF]
        rows_v = rest[NBUF:2 * NBUF]
        sem_g = rest[2 * NBUF:3 * NBUF]
        sem_o = rest[3 * NBUF:4 * NBUF]
        sem_i = rest[4 * NBUF:5 * NBUF]
        _sc_body(actions_hbm, table_hbm, out_hbm, table_sh,
                 idx_v, rows_v, sem_g, sem_o, sem_i)

    scratch = [pltpu.VMEM_SHARED((8, EMBED_DIM), jnp.float32)]
    scratch += [pltpu.VMEM((NB, HIST), jnp.int32) for _ in range(NBUF)]
    scratch += [pltpu.VMEM((NB, HIST, EMBED_DIM), jnp.float32)
                for _ in range(NBUF)]
    scratch += [pltpu.SemaphoreType.DMA for _ in range(3 * NBUF)]

    return pl.kernel(
        body,
        out_type=jax.ShapeDtypeStruct((BATCH, HIST, EMBED_DIM), jnp.float32),
        mesh=mesh,
        scratch_types=scratch,
        compiler_params=pltpu.CompilerParams(use_tc_tiling_on_sc=False),
    )(actions, table8)


def kernel(actions, embedding_weight):
    table8 = jnp.concatenate(
        [jnp.zeros((1, EMBED_DIM), jnp.float32), embedding_weight], axis=0)
    return _embed_sc(actions, table8)
